# baseline (speedup 1.0000x reference)
import functools

import jax
import jax.numpy as jnp
from jax import lax
from jax.experimental import pallas as pl
from jax.experimental.pallas import tpu as pltpu
from jax.experimental.pallas import tpu_sc as plsc

B = 1024
D = 128
C = 100000
P = 20

_NC = 2
_NS = 16
_NW = _NC * _NS
_NPW = (B * P) // _NW
_NCH = _NPW // 128
_TPW = B // _NW

_TCOL = 3584
_NSTEP = (C + _TCOL - 1) // _TCOL


def _sc_rows_kernel(v_hbm, l_hbm, neg_hbm, tgt_hbm, s_hbm, pf_hbm, nf_hbm,
                    gv_hbm, gt_hbm, pm_hbm, nm_hbm,
                    idx_v, nclu_v, rows_v, trows_v, tgt_v, pidx_v, val_v,
                    sem, sem2, sem3, sem4):
    wid = lax.axis_index("s") * _NC + lax.axis_index("c")
    base = wid * _NPW

    pltpu.sync_copy(tgt_hbm.at[wid], tgt_v)
    tgt_gather = pltpu.async_copy(v_hbm.at[tgt_v], trows_v, sem3)

    def pick(src, dst):
        pltpu.sync_copy(src.at[wid], pidx_v)
        descs = [
            pltpu.async_copy(s_hbm.at[pidx_v.at[j]], val_v.at[j], sem4)
            for j in range(_NCH)
        ]
        for d in descs:
            d.wait()
        pltpu.sync_copy(val_v, dst.at[wid])

    pltpu.sync_copy(neg_hbm.at[wid], idx_v)
    l_descs = [
        pltpu.async_copy(l_hbm.at[idx_v.at[j]], nclu_v.at[j], sem)
        for j in range(_NCH)
    ]
    v_descs = []
    for j in range(_NCH):
        l_descs[j].wait()
        v_descs.append(
            pltpu.async_copy(v_hbm.at[nclu_v.at[j]],
                             rows_v.at[pl.ds(j * 128, 128)], sem))
    out_descs = []
    for j in range(_NCH):
        v_descs[j].wait()
        out_descs.append(
            pltpu.async_copy(rows_v.at[pl.ds(j * 128, 128)],
                             gv_hbm.at[pl.ds(base + j * 128, 128)], sem2))
    pick(pf_hbm, pm_hbm)
    pick(nf_hbm, nm_hbm)
    tgt_gather.wait()
    pltpu.async_copy(trows_v, gt_hbm.at[pl.ds(wid * _TPW, _TPW)], sem3).wait()
    for dsc in out_descs:
        dsc.wait()


@functools.cache
def _sc_rows():
    return functools.partial(
        pl.kernel,
        mesh=plsc.VectorSubcoreMesh(core_axis_name="c", subcore_axis_name="s"),
        out_type=(
            jax.ShapeDtypeStruct((B * P, D), jnp.float32),
            jax.ShapeDtypeStruct((B, D), jnp.float32),
            jax.ShapeDtypeStruct((_NW, _NCH, 128), jnp.float32),
            jax.ShapeDtypeStruct((_NW, _NCH, 128), jnp.float32),
        ),
        scratch_types=[
            pltpu.VMEM((_NCH, 128), jnp.int32),
            pltpu.VMEM((_NCH, 128), jnp.int32),
            pltpu.VMEM((_NPW, D), jnp.float32),
            pltpu.VMEM((_TPW, D), jnp.float32),
            pltpu.VMEM((_TPW,), jnp.int32),
            pltpu.VMEM((_NCH, 128), jnp.int32),
            pltpu.VMEM((_NCH, 128), jnp.float32),
            pltpu.SemaphoreType.DMA,
            pltpu.SemaphoreType.DMA,
            pltpu.SemaphoreType.DMA,
            pltpu.SemaphoreType.DMA,
        ],
    )(_sc_rows_kernel)


def _sims_body(x_ref, s_ref):
    x = x_ref[...]
    norm = jnp.sqrt(jnp.sum(x * x, axis=1, keepdims=True))
    xn = x / (norm + 1e-12)
    s_ref[...] = lax.dot_general(xn, xn, (((1,), (1,)), ((), ())),
                                 preferred_element_type=jnp.float32)


def _sims_call(inputs):
    return pl.pallas_call(
        _sims_body,
        out_shape=jax.ShapeDtypeStruct((B, B), jnp.float32),
    )(inputs)


def _mm_body(x_ref, v_ref, gt_ref, gv_ref, pm_ref, nm_ref, pos_ref, neg_ref,
             out_ref, loss_ref, s_ref, xn_ref, nthr_ref, ntmax_ref, hn_sc):
    k = pl.program_id(0)
    x = x_ref[...]
    v = v_ref[...]
    logits = lax.dot_general(x, v, (((1,), (1,)), ((), ())),
                             preferred_element_type=jnp.float32)
    out_ref[...] = logits

    @pl.when(k == 0)
    def _init():
        s_ref[...] = jnp.zeros((B, 1), jnp.float32)
        norm = jnp.sqrt(jnp.sum(x * x, axis=1, keepdims=True))
        xn_ref[...] = x / (norm + 1e-12)
        gt = gt_ref[...]
        tdot = jnp.sum(x * gt, axis=1, keepdims=True)
        psim_t = tdot / (norm + 1e-12)
        pt_mask = psim_t != 0.0
        has_p = pos_ref[...] < B
        pmin = jnp.minimum(
            jnp.min(jnp.where(has_p, pm_ref[...], 3.0), axis=1, keepdims=True),
            jnp.where(pt_mask, psim_t, 3.0))
        nthr_ref[...] = pmin - 0.1
        ntmax_ref[...] = jnp.full((B, 1), -3.0, jnp.float32)
        hn_sc[0] = 0.0

    @pl.when(k < P)
    def _nsim_t_step():
        gvk = gv_ref[0]
        val = jnp.sum(xn_ref[...] * gvk, axis=1, keepdims=True)
        mask = val != 0.0
        ntmax_ref[...] = jnp.maximum(ntmax_ref[...],
                                     jnp.where(mask, val, -3.0))
        hmask = mask & (val > nthr_ref[...]) & (val < 0.999999)
        hn_sc[0] += jnp.sum(jnp.where(hmask,
                                      jnp.exp(50.0 * (val - 0.5)), 0.0))

    @pl.when(k < _NSTEP - 1)
    def _acc():
        s_ref[...] += jnp.sum(jnp.exp(logits), axis=1, keepdims=True)

    @pl.when(k == _NSTEP - 1)
    def _fin():
        col = k * _TCOL + lax.broadcasted_iota(jnp.int32, (B, _TCOL), 1)
        lv = jnp.where(col < C, logits, -jnp.inf)
        s = s_ref[...] + jnp.sum(jnp.exp(lv), axis=1, keepdims=True)
        lse = jnp.log(s)

        norm = jnp.sqrt(jnp.sum(x * x, axis=1, keepdims=True))
        gt = gt_ref[...]
        tdot = jnp.sum(x * gt, axis=1, keepdims=True)
        bu = jnp.sum(lse - tdot) / B
        psim_t = tdot / (norm + 1e-12)
        pt_mask = psim_t != 0.0

        psim_m = pm_ref[...]
        nsim_m = nm_ref[...]
        has_p = pos_ref[...] < B
        has_n = neg_ref[...] < B

        nmax = jnp.maximum(
            jnp.max(jnp.where(has_n, nsim_m, -3.0), axis=1, keepdims=True),
            ntmax_ref[...])
        p_thrd = nmax + 0.1
        n_thrd = nthr_ref[...]

        hp_mask_m = has_p & (psim_m < p_thrd)
        hp_mask_t = pt_mask & (psim_t < p_thrd)
        hp = (jnp.sum(jnp.where(hp_mask_m,
                                jnp.exp(-2.0 * (psim_m - 0.5)), 0.0))
              + jnp.sum(jnp.where(hp_mask_t,
                                  jnp.exp(-2.0 * (psim_t - 0.5)), 0.0)))
        hn_mask_m = has_n & (nsim_m > n_thrd) & (nsim_m < 0.999999)
        hn = (hn_sc[0]
              + jnp.sum(jnp.where(hn_mask_m,
                                  jnp.exp(50.0 * (nsim_m - 0.5)), 0.0)))

        hpv = jnp.full((1, 128), hp, jnp.float32)
        hnv = jnp.full((1, 128), hn, jnp.float32)
        h = 0.5 * jnp.log(1.0 + hpv) + (1.0 / 50.0) * jnp.log(1.0 + hnv)
        loss_ref[...] = bu + 10.0 * h


def _mm_call(inputs, V, gt, gv, pm, nm, pos, neg):
    return pl.pallas_call(
        _mm_body,
        grid=(_NSTEP,),
        in_specs=[
            pl.BlockSpec((B, D), lambda k: (0, 0)),
            pl.BlockSpec((_TCOL, D), lambda k: (k, 0)),
            pl.BlockSpec((B, D), lambda k: (0, 0)),
            pl.BlockSpec((1, B, D), lambda k: (jnp.minimum(k, P - 1), 0, 0)),
            pl.BlockSpec((B, P), lambda k: (0, 0)),
            pl.BlockSpec((B, P), lambda k: (0, 0)),
            pl.BlockSpec((B, P), lambda k: (0, 0)),
            pl.BlockSpec((B, P), lambda k: (0, 0)),
        ],
        out_specs=[
            pl.BlockSpec((B, _TCOL), lambda k: (0, k)),
            pl.BlockSpec((1, 128), lambda k: (0, 0)),
        ],
        out_shape=[
            jax.ShapeDtypeStruct((B, C), jnp.float32),
            jax.ShapeDtypeStruct((1, 128), jnp.float32),
        ],
        scratch_shapes=[
            pltpu.VMEM((B, 1), jnp.float32),
            pltpu.VMEM((B, D), jnp.float32),
            pltpu.VMEM((B, 1), jnp.float32),
            pltpu.VMEM((B, 1), jnp.float32),
            pltpu.SMEM((1,), jnp.float32),
        ],
        compiler_params=pltpu.CompilerParams(
            dimension_semantics=("arbitrary",)),
    )(inputs, V, gt, gv, pm, nm, pos, neg)


def kernel(inputs, targets, label_to_pairs, indexs, all_label_to_clusterid,
           epoch, V):
    pos = label_to_pairs[:, 0, :].astype(jnp.int32)
    neg = label_to_pairs[:, 1, :].astype(jnp.int32)
    rowbase = jnp.arange(B, dtype=jnp.int32)[:, None] * B
    pf = (rowbase + jnp.minimum(pos, B - 1)).reshape(_NW, _NCH, 128)
    nf = (rowbase + jnp.minimum(neg, B - 1)).reshape(_NW, _NCH, 128)
    negr = neg.T.reshape(_NW, _NCH, 128)
    tgtr = targets.astype(jnp.int32).reshape(_NW, _TPW)

    sims = _sims_call(inputs)
    gv, gt, pm, nm = _sc_rows()(V, all_label_to_clusterid.astype(jnp.int32),
                                negr, tgtr, sims.reshape(B * B), pf, nf)
    outputs, lossv = _mm_call(inputs, V, gt, gv.reshape(P, B, D),
                              pm.reshape(B, P), nm.reshape(B, P), pos, neg)
    return lossv[0, 0], outputs


# speedup vs baseline: 1.0047x; 1.0047x over previous
import functools

import jax
import jax.numpy as jnp
from jax import lax
from jax.experimental import pallas as pl
from jax.experimental.pallas import tpu as pltpu
from jax.experimental.pallas import tpu_sc as plsc

B = 1024
D = 128
C = 100000
P = 20

_NC = 2
_NS = 16
_NW = _NC * _NS
_NPW = (B * P) // _NW
_NCH = _NPW // 128
_TPW = B // _NW

_TCOL = 3584
_NSTEP = (C + _TCOL - 1) // _TCOL


def _sc_rows_kernel(v_hbm, l_hbm, neg_hbm, tgt_hbm, gv_hbm, gt_hbm,
                    idx_v, nclu_v, rows_v, trows_v, tgt_v, sem, sem2, sem3):
    wid = lax.axis_index("s") * _NC + lax.axis_index("c")
    base = wid * _NPW

    pltpu.sync_copy(tgt_hbm.at[wid], tgt_v)
    tgt_gather = pltpu.async_copy(v_hbm.at[tgt_v], trows_v, sem3)

    pltpu.sync_copy(neg_hbm.at[wid], idx_v)
    l_descs = [
        pltpu.async_copy(l_hbm.at[idx_v.at[j]], nclu_v.at[j], sem)
        for j in range(_NCH)
    ]
    v_descs = []
    for j in range(_NCH):
        l_descs[j].wait()
        v_descs.append(
            pltpu.async_copy(v_hbm.at[nclu_v.at[j]],
                             rows_v.at[pl.ds(j * 128, 128)], sem))
    out_descs = []
    for j in range(_NCH):
        v_descs[j].wait()
        out_descs.append(
            pltpu.async_copy(rows_v.at[pl.ds(j * 128, 128)],
                             gv_hbm.at[pl.ds(base + j * 128, 128)], sem2))
    tgt_gather.wait()
    pltpu.async_copy(trows_v, gt_hbm.at[pl.ds(wid * _TPW, _TPW)], sem3).wait()
    for dsc in out_descs:
        dsc.wait()


@functools.cache
def _sc_rows():
    return functools.partial(
        pl.kernel,
        mesh=plsc.VectorSubcoreMesh(core_axis_name="c", subcore_axis_name="s"),
        out_type=(
            jax.ShapeDtypeStruct((B * P, D), jnp.float32),
            jax.ShapeDtypeStruct((B, D), jnp.float32),
        ),
        scratch_types=[
            pltpu.VMEM((_NCH, 128), jnp.int32),
            pltpu.VMEM((_NCH, 128), jnp.int32),
            pltpu.VMEM((_NPW, D), jnp.float32),
            pltpu.VMEM((_TPW, D), jnp.float32),
            pltpu.VMEM((_TPW,), jnp.int32),
            pltpu.SemaphoreType.DMA,
            pltpu.SemaphoreType.DMA,
            pltpu.SemaphoreType.DMA,
        ],
    )(_sc_rows_kernel)


def _sc_sims_kernel(s_hbm, pf_hbm, nf_hbm, pm_hbm, nm_hbm, idx_v, val_v, sem):
    wid = lax.axis_index("s") * _NC + lax.axis_index("c")

    def pick(src, dst):
        pltpu.sync_copy(src.at[wid], idx_v)
        descs = [
            pltpu.async_copy(s_hbm.at[idx_v.at[j]], val_v.at[j], sem)
            for j in range(_NCH)
        ]
        for d in descs:
            d.wait()
        pltpu.sync_copy(val_v, dst.at[wid])

    pick(pf_hbm, pm_hbm)
    pick(nf_hbm, nm_hbm)


@functools.cache
def _sc_sims():
    return functools.partial(
        pl.kernel,
        mesh=plsc.VectorSubcoreMesh(core_axis_name="c", subcore_axis_name="s"),
        out_type=(
            jax.ShapeDtypeStruct((_NW, _NCH, 128), jnp.float32),
            jax.ShapeDtypeStruct((_NW, _NCH, 128), jnp.float32),
        ),
        scratch_types=[
            pltpu.VMEM((_NCH, 128), jnp.int32),
            pltpu.VMEM((_NCH, 128), jnp.float32),
            pltpu.SemaphoreType.DMA,
        ],
    )(_sc_sims_kernel)


def _sims_body(x_ref, s_ref):
    x = x_ref[...]
    norm = jnp.sqrt(jnp.sum(x * x, axis=1, keepdims=True))
    xn = x / (norm + 1e-12)
    s_ref[...] = lax.dot_general(xn, xn, (((1,), (1,)), ((), ())),
                                 preferred_element_type=jnp.float32)


def _sims_call(inputs):
    return pl.pallas_call(
        _sims_body,
        out_shape=jax.ShapeDtypeStruct((B, B), jnp.float32),
    )(inputs)


def _mm_body(x_ref, v_ref, gt_ref, gv_ref, pm_ref, nm_ref, pos_ref, neg_ref,
             out_ref, loss_ref, s_ref, xn_ref, nthr_ref, ntmax_ref, hn_sc):
    k = pl.program_id(0)
    x = x_ref[...]
    v = v_ref[...]
    logits = lax.dot_general(x, v, (((1,), (1,)), ((), ())),
                             preferred_element_type=jnp.float32)
    out_ref[...] = logits

    @pl.when(k == 0)
    def _init():
        s_ref[...] = jnp.zeros((B, 1), jnp.float32)
        norm = jnp.sqrt(jnp.sum(x * x, axis=1, keepdims=True))
        xn_ref[...] = x / (norm + 1e-12)
        gt = gt_ref[...]
        tdot = jnp.sum(x * gt, axis=1, keepdims=True)
        psim_t = tdot / (norm + 1e-12)
        pt_mask = psim_t != 0.0
        has_p = pos_ref[...] < B
        pmin = jnp.minimum(
            jnp.min(jnp.where(has_p, pm_ref[...], 3.0), axis=1, keepdims=True),
            jnp.where(pt_mask, psim_t, 3.0))
        nthr_ref[...] = pmin - 0.1
        ntmax_ref[...] = jnp.full((B, 1), -3.0, jnp.float32)
        hn_sc[0] = 0.0

    @pl.when(k < P)
    def _nsim_t_step():
        gvk = gv_ref[0]
        val = jnp.sum(xn_ref[...] * gvk, axis=1, keepdims=True)
        mask = val != 0.0
        ntmax_ref[...] = jnp.maximum(ntmax_ref[...],
                                     jnp.where(mask, val, -3.0))
        hmask = mask & (val > nthr_ref[...]) & (val < 0.999999)
        hn_sc[0] += jnp.sum(jnp.where(hmask,
                                      jnp.exp(50.0 * (val - 0.5)), 0.0))

    @pl.when(k < _NSTEP - 1)
    def _acc():
        s_ref[...] += jnp.sum(jnp.exp(logits), axis=1, keepdims=True)

    @pl.when(k == _NSTEP - 1)
    def _fin():
        col = k * _TCOL + lax.broadcasted_iota(jnp.int32, (B, _TCOL), 1)
        lv = jnp.where(col < C, logits, -jnp.inf)
        s = s_ref[...] + jnp.sum(jnp.exp(lv), axis=1, keepdims=True)
        lse = jnp.log(s)

        norm = jnp.sqrt(jnp.sum(x * x, axis=1, keepdims=True))
        gt = gt_ref[...]
        tdot = jnp.sum(x * gt, axis=1, keepdims=True)
        bu = jnp.sum(lse - tdot) / B
        psim_t = tdot / (norm + 1e-12)
        pt_mask = psim_t != 0.0

        psim_m = pm_ref[...]
        nsim_m = nm_ref[...]
        has_p = pos_ref[...] < B
        has_n = neg_ref[...] < B

        nmax = jnp.maximum(
            jnp.max(jnp.where(has_n, nsim_m, -3.0), axis=1, keepdims=True),
            ntmax_ref[...])
        p_thrd = nmax + 0.1
        n_thrd = nthr_ref[...]

        hp_mask_m = has_p & (psim_m < p_thrd)
        hp_mask_t = pt_mask & (psim_t < p_thrd)
        hp = (jnp.sum(jnp.where(hp_mask_m,
                                jnp.exp(-2.0 * (psim_m - 0.5)), 0.0))
              + jnp.sum(jnp.where(hp_mask_t,
                                  jnp.exp(-2.0 * (psim_t - 0.5)), 0.0)))
        hn_mask_m = has_n & (nsim_m > n_thrd) & (nsim_m < 0.999999)
        hn = (hn_sc[0]
              + jnp.sum(jnp.where(hn_mask_m,
                                  jnp.exp(50.0 * (nsim_m - 0.5)), 0.0)))

        hpv = jnp.full((1, 128), hp, jnp.float32)
        hnv = jnp.full((1, 128), hn, jnp.float32)
        h = 0.5 * jnp.log(1.0 + hpv) + (1.0 / 50.0) * jnp.log(1.0 + hnv)
        loss_ref[...] = bu + 10.0 * h


def _mm_call(inputs, V, gt, gv, pm, nm, pos, neg):
    return pl.pallas_call(
        _mm_body,
        grid=(_NSTEP,),
        in_specs=[
            pl.BlockSpec((B, D), lambda k: (0, 0)),
            pl.BlockSpec((_TCOL, D), lambda k: (k, 0)),
            pl.BlockSpec((B, D), lambda k: (0, 0)),
            pl.BlockSpec((1, B, D), lambda k: (jnp.minimum(k, P - 1), 0, 0)),
            pl.BlockSpec((B, P), lambda k: (0, 0)),
            pl.BlockSpec((B, P), lambda k: (0, 0)),
            pl.BlockSpec((B, P), lambda k: (0, 0)),
            pl.BlockSpec((B, P), lambda k: (0, 0)),
        ],
        out_specs=[
            pl.BlockSpec((B, _TCOL), lambda k: (0, k)),
            pl.BlockSpec((1, 128), lambda k: (0, 0)),
        ],
        out_shape=[
            jax.ShapeDtypeStruct((B, C), jnp.float32),
            jax.ShapeDtypeStruct((1, 128), jnp.float32),
        ],
        scratch_shapes=[
            pltpu.VMEM((B, 1), jnp.float32),
            pltpu.VMEM((B, D), jnp.float32),
            pltpu.VMEM((B, 1), jnp.float32),
            pltpu.VMEM((B, 1), jnp.float32),
            pltpu.SMEM((1,), jnp.float32),
        ],
        compiler_params=pltpu.CompilerParams(
            dimension_semantics=("arbitrary",)),
    )(inputs, V, gt, gv, pm, nm, pos, neg)


def kernel(inputs, targets, label_to_pairs, indexs, all_label_to_clusterid,
           epoch, V):
    pos = label_to_pairs[:, 0, :].astype(jnp.int32)
    neg = label_to_pairs[:, 1, :].astype(jnp.int32)
    rowbase = jnp.arange(B, dtype=jnp.int32)[:, None] * B
    pf = (rowbase + jnp.minimum(pos, B - 1)).reshape(_NW, _NCH, 128)
    nf = (rowbase + jnp.minimum(neg, B - 1)).reshape(_NW, _NCH, 128)
    negr = neg.T.reshape(_NW, _NCH, 128)
    tgtr = targets.astype(jnp.int32).reshape(_NW, _TPW)

    gv, gt = _sc_rows()(V, all_label_to_clusterid.astype(jnp.int32),
                        negr, tgtr)
    sims = _sims_call(inputs)
    pm, nm = _sc_sims()(sims.reshape(B * B), pf, nf)
    outputs, lossv = _mm_call(inputs, V, gt, gv.reshape(P, B, D),
                              pm.reshape(B, P), nm.reshape(B, P), pos, neg)
    return lossv[0, 0], outputs
